# baseline (speedup 1.0000x reference)
import functools

import jax
import jax.numpy as jnp
from jax import lax
from jax.experimental import pallas as pl
from jax.experimental.pallas import tpu as pltpu
from jax.experimental.pallas import tpu_sc as plsc

DIM = 64
BATCH = 4096
NC = 2
NS = 16
L = 16
NW = NC * NS
BPW = BATCH // NW


def _sc_kernel(idx_hbm, u_hbm, table_hbm, w_hbm, b_hbm, out_hbm,
               idx_v, rows_v, u_v, w_v, b_v, out_v, sem):
    wid = lax.axis_index("s") * NC + lax.axis_index("c")
    base = wid * BPW

    pltpu.sync_copy(idx_hbm.at[pl.ds(base, BPW)], idx_v)
    pltpu.sync_copy(u_hbm, u_v)
    pltpu.sync_copy(w_hbm, w_v)
    pltpu.sync_copy(b_hbm, b_v)

    gather = pltpu.async_copy(table_hbm.at[idx_v], rows_v, sem)

    lane = lax.iota(jnp.int32, L)
    b_hold = b_v[pl.ds(0, L)][0]

    p = jnp.zeros((L,), jnp.float32)
    for k in range(DIM // L):
        p = p + u_v[pl.ds(k * L, L)] * w_v[pl.ds(k * L, L)]
    for shift in (8, 4, 2, 1):
        b_v[pl.ds(0, L)] = p
        p = p + plsc.load_gather(b_v, [lane ^ shift])
    c = p[0] + b_hold

    w2 = [w_v[pl.ds(DIM + k * L, L)] for k in range(DIM // L)]

    gather.wait()

    def group(g, carry):
        row_ids = g * L + lane
        acc = jnp.full((L,), c, dtype=jnp.float32)
        for j in range(DIM):
            col = plsc.load_gather(rows_v, [row_ids, jnp.full((L,), j, jnp.int32)])
            acc = acc + col * w2[j // L][j % L]
        out_v[pl.ds(g * L, L)] = acc
        return carry

    lax.fori_loop(0, BPW // L, group, 0)

    pltpu.sync_copy(out_v, out_hbm.at[pl.ds(base, BPW)])


@jax.jit
def kernel(item_idx, user_emb, item_emb, fc_w, fc_b):
    mesh = plsc.VectorSubcoreMesh(core_axis_name="c", subcore_axis_name="s")
    run = functools.partial(
        pl.kernel,
        mesh=mesh,
        compiler_params=pltpu.CompilerParams(needs_layout_passes=False,
                                             use_tc_tiling_on_sc=False),
        out_type=jax.ShapeDtypeStruct((BATCH,), jnp.float32),
        scratch_types=[
            pltpu.VMEM((BPW,), jnp.int32),
            pltpu.VMEM((BPW, DIM), jnp.float32),
            pltpu.VMEM((DIM,), jnp.float32),
            pltpu.VMEM((2 * DIM,), jnp.float32),
            pltpu.VMEM((L,), jnp.float32),
            pltpu.VMEM((BPW,), jnp.float32),
            pltpu.SemaphoreType.DMA,
        ],
    )(_sc_kernel)
    out = run(item_idx.astype(jnp.int32), user_emb.reshape(DIM),
              item_emb, fc_w.reshape(2 * DIM),
              jnp.pad(fc_b, (0, L - 1)))
    return out.reshape(BATCH, 1)
